# baseline (speedup 1.0000x reference)
import functools

import jax
import jax.numpy as jnp
from jax import lax
from jax.experimental import pallas as pl
from jax.experimental.pallas import tpu as pltpu
from jax.experimental.pallas import tpu_sc as plsc

VOCAB = 100000
DIM = 128
BATCH = 4096
SEQ = 50

NC = 2
NS = 16
NW = NC * NS
B_PER_W = BATCH // NW
CHUNK = 4
NCHUNK = B_PER_W // CHUNK
LANES = 16
NV = DIM // LANES


def _sc_gather_sum(idx, table):
    mesh = plsc.VectorSubcoreMesh(core_axis_name="c", subcore_axis_name="s")

    @functools.partial(
        pl.kernel,
        mesh=mesh,
        out_type=jax.ShapeDtypeStruct((BATCH, DIM), jnp.float32),
        scratch_types=[
            pltpu.VMEM((B_PER_W, SEQ), jnp.int32),
            pltpu.VMEM((CHUNK, SEQ, DIM), jnp.float32),
            pltpu.VMEM((CHUNK, SEQ, DIM), jnp.float32),
            pltpu.VMEM((CHUNK, DIM), jnp.float32),
            pltpu.SemaphoreType.DMA,
            pltpu.SemaphoreType.DMA,
        ],
    )
    def k(idx_hbm, table_hbm, out_hbm, idx_v, rows0, rows1, acc_v,
          sem0, sem1):
        wid = lax.axis_index("s") * NC + lax.axis_index("c")
        ebase = wid * B_PER_W
        pltpu.sync_copy(idx_hbm.at[pl.ds(ebase, B_PER_W)], idx_v)

        def issue(c, buf, sem):
            for b in range(CHUNK):
                pltpu.async_copy(
                    table_hbm.at[idx_v.at[c * CHUNK + b]],
                    buf.at[b], sem)

        def drain(buf, sem):
            for b in range(CHUNK):
                pltpu.make_async_copy(
                    table_hbm.at[idx_v.at[0]], buf.at[b], sem).wait()

        def compute(c, buf):
            for b in range(CHUNK):
                def s_body(s, accs, b=b):
                    return tuple(a + buf[b, s, pl.ds(LANES * v, LANES)]
                                 for v, a in enumerate(accs))
                accs = lax.fori_loop(
                    0, SEQ, s_body,
                    tuple(jnp.zeros((LANES,), jnp.float32)
                          for _ in range(NV)),
                    unroll=5)
                for v, a in enumerate(accs):
                    acc_v[b, pl.ds(LANES * v, LANES)] = a
            obase = pl.multiple_of(ebase + c * CHUNK, CHUNK)
            pltpu.sync_copy(acc_v, out_hbm.at[pl.ds(obase, CHUNK)])

        issue(0, rows0, sem0)

        def pair_body(i, carry):
            c0 = i * 2
            c1 = c0 + 1
            issue(c1, rows1, sem1)
            drain(rows0, sem0)
            compute(c0, rows0)

            @pl.when(i + 1 < NCHUNK // 2)
            def _():
                issue(c1 + 1, rows0, sem0)

            drain(rows1, sem1)
            compute(c1, rows1)
            return carry

        lax.fori_loop(0, NCHUNK // 2, pair_body, 0)

    return k(idx, table)


def _mlp_body(s_ref, w1_ref, b1_ref, w2_ref, b2_ref, w3_ref, b3_ref, o_ref):
    dn = (((1,), (1,)), ((), ()))
    x = s_ref[...] * (1.0 / SEQ)
    h = jnp.tanh(lax.dot_general(x, w1_ref[...], dn,
                                 preferred_element_type=jnp.float32)
                 + b1_ref[...])
    h = jnp.tanh(lax.dot_general(h, w2_ref[...], dn,
                                 preferred_element_type=jnp.float32)
                 + b2_ref[...])
    z = jnp.tanh(lax.dot_general(h, w3_ref[...], dn,
                                 preferred_element_type=jnp.float32)
                 + b3_ref[...])
    a = z[:, 0:1]
    b = z[:, 1:2]
    lse = jnp.logaddexp(a, b)
    o_ref[...] = jnp.concatenate([a - lse, b - lse], axis=1)


def _tc_mlp(sums, w1, b1, w2, b2, w3, b3):
    blk = 512
    grid = BATCH // blk
    return pl.pallas_call(
        _mlp_body,
        grid=(grid,),
        in_specs=[
            pl.BlockSpec((blk, DIM), lambda i: (i, 0)),
            pl.BlockSpec(w1.shape, lambda i: (0, 0)),
            pl.BlockSpec(b1.shape, lambda i: (0, 0)),
            pl.BlockSpec(w2.shape, lambda i: (0, 0)),
            pl.BlockSpec(b2.shape, lambda i: (0, 0)),
            pl.BlockSpec(w3.shape, lambda i: (0, 0)),
            pl.BlockSpec(b3.shape, lambda i: (0, 0)),
        ],
        out_specs=pl.BlockSpec((blk, 2), lambda i: (i, 0)),
        out_shape=jax.ShapeDtypeStruct((BATCH, 2), jnp.float32),
    )(sums, w1, b1, w2, b2, w3, b3)


def kernel(input, emb_weight, out_w, out_b, out1_w, out1_b, out2_w, out2_b):
    sums = _sc_gather_sum(input, emb_weight)
    return _tc_mlp(sums,
                   out_w, out_b.reshape(1, -1),
                   out1_w, out1_b.reshape(1, -1),
                   out2_w, out2_b.reshape(1, -1))


# speedup vs baseline: 1.0760x; 1.0760x over previous
import functools

import jax
import jax.numpy as jnp
from jax import lax
from jax.experimental import pallas as pl
from jax.experimental.pallas import tpu as pltpu
from jax.experimental.pallas import tpu_sc as plsc

VOCAB = 100000
DIM = 128
BATCH = 4096
SEQ = 50

NC = 2
NS = 16
NW = NC * NS
B_PER_W = BATCH // NW
CHUNK = 8
ROWS = CHUNK * SEQ
NCHUNK = B_PER_W // CHUNK
LANES = 16
NV = DIM // LANES


def _sc_gather_sum(idx_flat, table):
    mesh = plsc.VectorSubcoreMesh(core_axis_name="c", subcore_axis_name="s")

    @functools.partial(
        pl.kernel,
        mesh=mesh,
        out_type=jax.ShapeDtypeStruct((BATCH, DIM), jnp.float32),
        scratch_types=[
            pltpu.VMEM((B_PER_W * SEQ,), jnp.int32),
            pltpu.VMEM((ROWS, DIM), jnp.float32),
            pltpu.VMEM((ROWS, DIM), jnp.float32),
            pltpu.VMEM((CHUNK, DIM), jnp.float32),
            pltpu.VMEM((CHUNK, DIM), jnp.float32),
            pltpu.SemaphoreType.DMA,
            pltpu.SemaphoreType.DMA,
            pltpu.SemaphoreType.DMA,
            pltpu.SemaphoreType.DMA,
        ],
    )
    def k(idx_hbm, table_hbm, out_hbm, idx_v, rows0, rows1, acc0, acc1,
          sem0, sem1, osem0, osem1):
        wid = lax.axis_index("s") * NC + lax.axis_index("c")
        ibase = wid * (B_PER_W * SEQ)
        pltpu.sync_copy(idx_hbm.at[pl.ds(ibase, B_PER_W * SEQ)], idx_v)

        parts = ((0, 128), (128, 128), (256, 128), (384, 16))

        def issue(c, buf, sem):
            off = pl.multiple_of(c * ROWS, 8)
            for lo, sz in parts:
                pltpu.async_copy(
                    table_hbm.at[idx_v.at[pl.ds(off + lo, sz)]],
                    buf.at[pl.ds(lo, sz)], sem)

        def drain(buf, sem):
            for lo, sz in parts:
                pltpu.make_async_copy(
                    table_hbm.at[idx_v.at[pl.ds(lo, sz)]],
                    buf.at[pl.ds(lo, sz)], sem).wait()

        def out_wait(acc, osem):
            pltpu.make_async_copy(
                acc, out_hbm.at[pl.ds(0, CHUNK)], osem).wait()

        def compute(c, buf, acc, osem):
            for b in range(CHUNK):
                def s_body(s, accs, b=b):
                    r = b * SEQ + s
                    return tuple(a + buf[r, pl.ds(LANES * v, LANES)]
                                 for v, a in enumerate(accs))
                accs = lax.fori_loop(
                    0, SEQ, s_body,
                    tuple(jnp.zeros((LANES,), jnp.float32)
                          for _ in range(NV)),
                    unroll=5)
                for v, a in enumerate(accs):
                    acc[b, pl.ds(LANES * v, LANES)] = a
            obase = pl.multiple_of(wid * B_PER_W + c * CHUNK, 8)
            pltpu.async_copy(acc, out_hbm.at[pl.ds(obase, CHUNK)], osem)

        issue(0, rows0, sem0)

        def pair_body(i, carry):
            c0 = i * 2
            c1 = c0 + 1
            issue(c1, rows1, sem1)
            drain(rows0, sem0)

            @pl.when(i > 0)
            def _():
                out_wait(acc0, osem0)

            compute(c0, rows0, acc0, osem0)

            @pl.when(i + 1 < NCHUNK // 2)
            def _():
                issue(c1 + 1, rows0, sem0)

            drain(rows1, sem1)

            @pl.when(i > 0)
            def _():
                out_wait(acc1, osem1)

            compute(c1, rows1, acc1, osem1)
            return carry

        lax.fori_loop(0, NCHUNK // 2, pair_body, 0)
        out_wait(acc0, osem0)
        out_wait(acc1, osem1)

    return k(idx_flat, table)


def _mlp_body(s_ref, w1_ref, b1_ref, w2_ref, b2_ref, w3_ref, b3_ref, o_ref):
    dn = (((1,), (1,)), ((), ()))
    x = s_ref[...] * (1.0 / SEQ)
    h = jnp.tanh(lax.dot_general(x, w1_ref[...], dn,
                                 preferred_element_type=jnp.float32)
                 + b1_ref[...])
    h = jnp.tanh(lax.dot_general(h, w2_ref[...], dn,
                                 preferred_element_type=jnp.float32)
                 + b2_ref[...])
    z = jnp.tanh(lax.dot_general(h, w3_ref[...], dn,
                                 preferred_element_type=jnp.float32)
                 + b3_ref[...])
    a = z[:, 0:1]
    b = z[:, 1:2]
    lse = jnp.logaddexp(a, b)
    o_ref[...] = jnp.concatenate([a - lse, b - lse], axis=1)


def _tc_mlp(sums, w1, b1, w2, b2, w3, b3):
    blk = 2048
    grid = BATCH // blk
    return pl.pallas_call(
        _mlp_body,
        grid=(grid,),
        in_specs=[
            pl.BlockSpec((blk, DIM), lambda i: (i, 0)),
            pl.BlockSpec(w1.shape, lambda i: (0, 0)),
            pl.BlockSpec(b1.shape, lambda i: (0, 0)),
            pl.BlockSpec(w2.shape, lambda i: (0, 0)),
            pl.BlockSpec(b2.shape, lambda i: (0, 0)),
            pl.BlockSpec(w3.shape, lambda i: (0, 0)),
            pl.BlockSpec(b3.shape, lambda i: (0, 0)),
        ],
        out_specs=pl.BlockSpec((blk, 2), lambda i: (i, 0)),
        out_shape=jax.ShapeDtypeStruct((BATCH, 2), jnp.float32),
    )(sums, w1, b1, w2, b2, w3, b3)


def kernel(input, emb_weight, out_w, out_b, out1_w, out1_b, out2_w, out2_b):
    sums = _sc_gather_sum(input.reshape(-1), emb_weight)
    return _tc_mlp(sums,
                   out_w, out_b.reshape(1, -1),
                   out1_w, out1_b.reshape(1, -1),
                   out2_w, out2_b.reshape(1, -1))


# speedup vs baseline: 1.1693x; 1.0867x over previous
import functools

import jax
import jax.numpy as jnp
from jax import lax
from jax.experimental import pallas as pl
from jax.experimental.pallas import tpu as pltpu
from jax.experimental.pallas import tpu_sc as plsc

VOCAB = 100000
DIM = 128
BATCH = 4096
SEQ = 50

NC = 2
NS = 16
NW = NC * NS
B_PER_W = BATCH // NW
CHUNK = 4
ROWS = CHUNK * SEQ
NCHUNK = B_PER_W // CHUNK
DEPTH = 4
LANES = 16
NV = DIM // LANES

PARTS = ((0, 128), (128, ROWS - 128))


def _sc_gather_sum(idx_flat, table):
    mesh = plsc.VectorSubcoreMesh(core_axis_name="c", subcore_axis_name="s")

    @functools.partial(
        pl.kernel,
        mesh=mesh,
        out_type=jax.ShapeDtypeStruct((BATCH, DIM), jnp.float32),
        scratch_types=[
            pltpu.VMEM((B_PER_W * SEQ,), jnp.int32),
            *[pltpu.VMEM((ROWS, DIM), jnp.float32) for _ in range(DEPTH)],
            *[pltpu.VMEM((CHUNK, DIM), jnp.float32) for _ in range(DEPTH)],
            *[pltpu.SemaphoreType.DMA for _ in range(2 * DEPTH)],
        ],
    )
    def k(idx_hbm, table_hbm, out_hbm, idx_v, *rest):
        bufs = rest[:DEPTH]
        accs = rest[DEPTH:2 * DEPTH]
        sems = rest[2 * DEPTH:3 * DEPTH]
        osems = rest[3 * DEPTH:4 * DEPTH]

        wid = lax.axis_index("s") * NC + lax.axis_index("c")
        ibase = wid * (B_PER_W * SEQ)
        pltpu.sync_copy(idx_hbm.at[pl.ds(ibase, B_PER_W * SEQ)], idx_v)

        def issue(c, buf, sem):
            off = pl.multiple_of(c * ROWS, 8)
            for lo, sz in PARTS:
                pltpu.async_copy(
                    table_hbm.at[idx_v.at[pl.ds(off + lo, sz)]],
                    buf.at[pl.ds(lo, sz)], sem)

        def drain(buf, sem):
            for lo, sz in PARTS:
                pltpu.make_async_copy(
                    table_hbm.at[idx_v.at[pl.ds(lo, sz)]],
                    buf.at[pl.ds(lo, sz)], sem).wait()

        def out_wait(acc, osem):
            pltpu.make_async_copy(
                acc, out_hbm.at[pl.ds(0, CHUNK)], osem).wait()

        def compute(c, buf, acc, osem):
            for b in range(CHUNK):
                def s_body(s, vacc, b=b):
                    r = b * SEQ + s
                    return tuple(a + buf[r, pl.ds(LANES * v, LANES)]
                                 for v, a in enumerate(vacc))
                vacc = lax.fori_loop(
                    0, SEQ, s_body,
                    tuple(jnp.zeros((LANES,), jnp.float32)
                          for _ in range(NV)),
                    unroll=5)
                for v, a in enumerate(vacc):
                    acc[b, pl.ds(LANES * v, LANES)] = a
            obase = pl.multiple_of(wid * B_PER_W + c * CHUNK, CHUNK)
            pltpu.async_copy(acc, out_hbm.at[pl.ds(obase, CHUNK)], osem)

        for j in range(DEPTH - 1):
            issue(j, bufs[j], sems[j])

        def ring_body(i, carry):
            for j in range(DEPTH):
                c = i * DEPTH + j
                drain(bufs[j], sems[j])

                @pl.when(i > 0)
                def _(j=j):
                    out_wait(accs[j], osems[j])

                @pl.when(c + DEPTH - 1 < NCHUNK)
                def _(c=c, j=j):
                    issue(c + DEPTH - 1, bufs[(j + DEPTH - 1) % DEPTH],
                          sems[(j + DEPTH - 1) % DEPTH])

                compute(c, bufs[j], accs[j], osems[j])
            return carry

        lax.fori_loop(0, NCHUNK // DEPTH, ring_body, 0)
        for j in range(DEPTH):
            out_wait(accs[j], osems[j])

    return k(idx_flat, table)


def _mlp_body(s_ref, w1_ref, b1_ref, w2_ref, b2_ref, w3_ref, b3_ref, o_ref):
    dn = (((1,), (1,)), ((), ()))
    x = s_ref[...] * (1.0 / SEQ)
    h = jnp.tanh(lax.dot_general(x, w1_ref[...], dn,
                                 preferred_element_type=jnp.float32)
                 + b1_ref[...])
    h = jnp.tanh(lax.dot_general(h, w2_ref[...], dn,
                                 preferred_element_type=jnp.float32)
                 + b2_ref[...])
    z = jnp.tanh(lax.dot_general(h, w3_ref[...], dn,
                                 preferred_element_type=jnp.float32)
                 + b3_ref[...])
    a = z[:, 0:1]
    b = z[:, 1:2]
    lse = jnp.logaddexp(a, b)
    o_ref[...] = jnp.concatenate([a - lse, b - lse], axis=1)


def _tc_mlp(sums, w1, b1, w2, b2, w3, b3):
    blk = 2048
    grid = BATCH // blk
    return pl.pallas_call(
        _mlp_body,
        grid=(grid,),
        in_specs=[
            pl.BlockSpec((blk, DIM), lambda i: (i, 0)),
            pl.BlockSpec(w1.shape, lambda i: (0, 0)),
            pl.BlockSpec(b1.shape, lambda i: (0, 0)),
            pl.BlockSpec(w2.shape, lambda i: (0, 0)),
            pl.BlockSpec(b2.shape, lambda i: (0, 0)),
            pl.BlockSpec(w3.shape, lambda i: (0, 0)),
            pl.BlockSpec(b3.shape, lambda i: (0, 0)),
        ],
        out_specs=pl.BlockSpec((blk, 2), lambda i: (i, 0)),
        out_shape=jax.ShapeDtypeStruct((BATCH, 2), jnp.float32),
    )(sums, w1, b1, w2, b2, w3, b3)


def kernel(input, emb_weight, out_w, out_b, out1_w, out1_b, out2_w, out2_b):
    sums = _sc_gather_sum(input.reshape(-1), emb_weight)
    return _tc_mlp(sums,
                   out_w, out_b.reshape(1, -1),
                   out1_w, out1_b.reshape(1, -1),
                   out2_w, out2_b.reshape(1, -1))
